# baseline (speedup 1.0000x reference)
import functools

import jax
import jax.numpy as jnp
from jax import lax
from jax.experimental import pallas as pl
from jax.experimental.pallas import tpu as pltpu
from jax.experimental.pallas import tpu_sc as plsc

_B = 16384
_D = 128
_NW = 32
_ROWS_PER_W = _B // _NW
_BLK = 64
_NBLK = _ROWS_PER_W // _BLK
_NBUF = 4
_PAD = 17


def _sc_dots(u2, i2, j2, user_table, item_table):
    mesh = plsc.VectorSubcoreMesh(core_axis_name="c", subcore_axis_name="s")

    row_bufs = [pltpu.VMEM((_BLK, _D), jnp.float32) for _ in range(3 * _NBUF)]

    @functools.partial(
        pl.kernel,
        out_type=(
            jax.ShapeDtypeStruct((_B,), jnp.float32),
            jax.ShapeDtypeStruct((_B,), jnp.float32),
        ),
        mesh=mesh,
        scratch_types=[
            pltpu.VMEM((_NBLK, _BLK), jnp.int32),
            pltpu.VMEM((_NBLK, _BLK), jnp.int32),
            pltpu.VMEM((_NBLK, _BLK), jnp.int32),
        ] + row_bufs + [
            pltpu.VMEM((16 * _PAD,), jnp.float32),
            pltpu.VMEM((16 * _PAD,), jnp.float32),
            pltpu.VMEM((_ROWS_PER_W,), jnp.float32),
            pltpu.VMEM((_ROWS_PER_W,), jnp.float32),
        ] + [pltpu.SemaphoreType.DMA] * (_NBUF + 1),
        compiler_params=pltpu.CompilerParams(needs_layout_passes=False),
    )
    def k(u_hbm, i_hbm, j_hbm, ut_hbm, it_hbm, pi_hbm, pj_hbm,
          idx_u, idx_i, idx_j, *rest):
        bufs = rest[:3 * _NBUF]
        tb_i, tb_j, oi_v, oj_v = rest[3 * _NBUF:3 * _NBUF + 4]
        sems = rest[3 * _NBUF + 4:3 * _NBUF + 4 + _NBUF]
        osem = rest[3 * _NBUF + 4 + _NBUF]
        sets = tuple(
            (bufs[3 * s], bufs[3 * s + 1], bufs[3 * s + 2], sems[s])
            for s in range(_NBUF)
        )

        wid = lax.axis_index("s") * 2 + lax.axis_index("c")

        def idx_copies():
            return (
                pltpu.make_async_copy(
                    u_hbm.at[pl.ds(wid * _NBLK, _NBLK)], idx_u, osem),
                pltpu.make_async_copy(
                    i_hbm.at[pl.ds(wid * _NBLK, _NBLK)], idx_i, osem),
                pltpu.make_async_copy(
                    j_hbm.at[pl.ds(wid * _NBLK, _NBLK)], idx_j, osem),
            )

        for c in idx_copies():
            c.start()
        for c in idx_copies():
            c.wait()

        def copies(b, sub):
            ue_v, ie_v, je_v, sem = sets[sub]
            return (
                pltpu.make_async_copy(ut_hbm.at[idx_u.at[b]], ue_v, sem),
                pltpu.make_async_copy(it_hbm.at[idx_i.at[b]], ie_v, sem),
                pltpu.make_async_copy(it_hbm.at[idx_j.at[b]], je_v, sem),
            )

        for s in range(_NBUF - 1):
            for c in copies(s, s):
                c.start()

        lanes = lax.iota(jnp.int32, 16)
        lanes17 = lanes * _PAD
        zv = jnp.zeros((16,), jnp.float32)

        def quad_body(p, _):
            for sub in range(_NBUF):
                b = _NBUF * p + sub
                ue_v, ie_v, je_v, _sem = sets[sub]

                @pl.when(b + _NBUF - 1 < _NBLK)
                def _():
                    for c in copies(b + _NBUF - 1, (sub + _NBUF - 1) % _NBUF):
                        c.start()

                for c in copies(b, sub):
                    c.wait()

                def grp_body(g, _, ue_v=ue_v, ie_v=ie_v, je_v=je_v, b=b):
                    r0 = g * 16

                    def row_body(r, _):
                        acc_i = zv
                        acc_j = zv
                        for c in range(_D // 16):
                            ue = ue_v[r0 + r, pl.ds(c * 16, 16)]
                            ie = ie_v[r0 + r, pl.ds(c * 16, 16)]
                            je = je_v[r0 + r, pl.ds(c * 16, 16)]
                            acc_i = acc_i + ue * ie
                            acc_j = acc_j + ue * je
                        row_idx = lanes + r * _PAD
                        plsc.store_scatter(tb_i, [row_idx], acc_i)
                        plsc.store_scatter(tb_j, [row_idx], acc_j)
                        return 0

                    lax.fori_loop(0, 16, row_body, 0, unroll=8)
                    s_i = zv
                    s_j = zv
                    for c in range(16):
                        s_i = s_i + plsc.load_gather(tb_i, [lanes17 + c])
                        s_j = s_j + plsc.load_gather(tb_j, [lanes17 + c])
                    oi_v[pl.ds(b * _BLK + r0, 16)] = s_i
                    oj_v[pl.ds(b * _BLK + r0, 16)] = s_j
                    return 0

                lax.fori_loop(0, _BLK // 16, grp_body, 0)
            return 0

        lax.fori_loop(0, _NBLK // _NBUF, quad_body, 0)

        o1 = pltpu.make_async_copy(
            oi_v, pi_hbm.at[pl.ds(wid * _ROWS_PER_W, _ROWS_PER_W)], osem)
        o2 = pltpu.make_async_copy(
            oj_v, pj_hbm.at[pl.ds(wid * _ROWS_PER_W, _ROWS_PER_W)], osem)
        o1.start()
        o2.start()
        o1.wait()
        o2.wait()

    return k(u2, i2, j2, user_table, item_table)


def _tc_loss_body(pi_ref, pj_ref, out_ref):
    x = pi_ref[...]
    y = pj_ref[...]

    def softplus(t):
        return jnp.maximum(t, 0.0) + jnp.log1p(jnp.exp(-jnp.abs(t)))

    out_ref[0, 0] = jnp.sum(softplus(-x)) + jnp.sum(softplus(y))


def kernel(u, i, j, user_table, item_table):
    u2 = u.reshape(_NBLK * _NW, _BLK).astype(jnp.int32)
    i2 = i.reshape(_NBLK * _NW, _BLK).astype(jnp.int32)
    j2 = j.reshape(_NBLK * _NW, _BLK).astype(jnp.int32)
    pred_i, pred_j = _sc_dots(u2, i2, j2, user_table, item_table)

    loss = pl.pallas_call(
        _tc_loss_body,
        out_shape=jax.ShapeDtypeStruct((1, 1), jnp.float32),
        out_specs=pl.BlockSpec(memory_space=pltpu.SMEM),
    )(pred_i.reshape(128, 128), pred_j.reshape(128, 128))
    return loss[0, 0]
